# baseline (speedup 1.0000x reference)
import functools

import numpy as np
import jax
import jax.numpy as jnp
from jax.experimental import pallas as pl

_NUM_PROJS = 128


def _sobol2_np(n):
    bits = 30
    ms = [1]
    for k in range(1, bits):
        ms.append((2 * ms[k - 1]) ^ ms[k - 1])
    v0 = [1 << (bits - 1 - k) for k in range(bits)]
    v1 = [ms[k] << (bits - 1 - k) for k in range(bits)]
    x0, x1 = 0, 0
    out = np.zeros((n, 2), dtype=np.float64)
    for i in range(1, n):
        c = 0
        j = i - 1
        while j & 1:
            j >>= 1
            c += 1
        x0 ^= v0[c]
        x1 ^= v1[c]
        out[i, 0] = x0 / float(1 << bits)
        out[i, 1] = x1 / float(1 << bits)
    return out


def _theta_np(num_projs):
    net = _sobol2_np(num_projs)
    alpha = net[:, 0:1]
    tau = net[:, 1:2]
    r = 2.0 * np.sqrt(np.maximum(tau - tau ** 2, 0.0))
    theta = np.concatenate([
        r * np.cos(2.0 * np.pi * alpha),
        r * np.sin(2.0 * np.pi * alpha),
        1.0 - 2.0 * tau,
    ], axis=1)
    return theta.astype(np.float32)


_THETA = _theta_np(_NUM_PROJS)


def _bitonic_sort_axis0(a):
    n, c = a.shape
    k = 2
    while k <= n:
        j = k // 2
        while j >= 1:
            g = n // (2 * j)
            b = a.reshape(g, 2, j, c)
            lo = jnp.minimum(b[:, 0], b[:, 1])
            hi = jnp.maximum(b[:, 0], b[:, 1])
            if k < n:
                gi = jax.lax.broadcasted_iota(jnp.int32, (g, 1, 1), 0)
                asc = ((gi * (2 * j)) & k) == 0
                first = jnp.where(asc, lo, hi)
                second = jnp.where(asc, hi, lo)
            else:
                first, second = lo, hi
            a = jnp.stack([first, second], axis=1).reshape(n, c)
            j //= 2
        k *= 2
    return a


def _tc_body2(x_ref, y_ref, theta_ref, o_ref):
    th = theta_ref[...]
    kp = th.shape[1]
    cols = []
    for bb in range(2):
        x = x_ref[bb]
        y = y_ref[bb]
        xp = (x[:, 0:1] * th[0:1, :] + x[:, 1:2] * th[1:2, :]
              + x[:, 2:3] * th[2:3, :])
        yp = (y[:, 0:1] * th[0:1, :] + y[:, 1:2] * th[1:2, :]
              + y[:, 2:3] * th[2:3, :])
        cols += [xp, yp]
    a = jnp.concatenate(cols, axis=1)
    a = _bitonic_sort_axis0(a)
    for bb in range(2):
        d = a[:, 2 * kp * bb:2 * kp * bb + kp] \
            - a[:, 2 * kp * bb + kp:2 * kp * (bb + 1)]
        o_ref[bb, 0, :] = jnp.sum(d * d, axis=0)


def _swd_tc_part(x, y, theta_t):
    b, n, _ = x.shape
    kp = theta_t.shape[1]
    s = pl.pallas_call(
        _tc_body2,
        grid=(b // 2,),
        in_specs=[
            pl.BlockSpec((2, n, 3), lambda i: (i, 0, 0)),
            pl.BlockSpec((2, n, 3), lambda i: (i, 0, 0)),
            pl.BlockSpec((3, kp), lambda i: (0, 0)),
        ],
        out_specs=pl.BlockSpec((2, 1, kp), lambda i: (i, 0, 0)),
        out_shape=jax.ShapeDtypeStruct((b, 1, kp), jnp.float32),
    )(x, y, theta_t)
    return s[:, 0, :]


from jax import lax
from jax.experimental.pallas import tpu as pltpu
from jax.experimental.pallas import tpu_sc as plsc

_NC, _NS = 2, 16


def _asc(v):
    return lax.sort(v, dimension=0, is_stable=False)


def _desc(v):
    return plsc.sort_key_val(v, v, descending=True)[0]


def _cmpx(u, v):
    return jnp.minimum(u, v), jnp.maximum(u, v)


def _sc_pass_a(ra, rb, R):
    @plsc.parallel_loop(0, R // 4, unroll=1)
    def body(g):
        base = g * 4
        for ref in (ra, rb):
            u0 = _asc(ref[base, :])
            u1 = _desc(ref[base + 1, :])
            u2 = _asc(ref[base + 2, :])
            u3 = _desc(ref[base + 3, :])
            lo0, hi0 = _cmpx(u0, u1)
            lo2, hi2 = _cmpx(u2, u3)
            a0 = _asc(lo0)
            a1 = _asc(hi0)
            b0 = _desc(hi2)
            b1 = _desc(lo2)
            w0, w2 = _cmpx(a0, b0)
            w1, w3 = _cmpx(a1, b1)
            o0, o1 = _cmpx(w0, w1)
            o2, o3 = _cmpx(w2, w3)
            ref[base, :] = o0
            ref[base + 1, :] = o1
            ref[base + 2, :] = o2
            ref[base + 3, :] = o3


def _sc_pass_b(ra, rb, R):
    @plsc.parallel_loop(0, R // 8, unroll=1)
    def body(g):
        base = g * 8
        for ref in (ra, rb):
            a0 = _asc(ref[base + 0, :])
            a1 = _asc(ref[base + 1, :])
            a2 = _asc(ref[base + 2, :])
            a3 = _asc(ref[base + 3, :])
            b3 = _desc(ref[base + 4, :])
            b2 = _desc(ref[base + 5, :])
            b1 = _desc(ref[base + 6, :])
            b0 = _desc(ref[base + 7, :])
            w0, w4 = _cmpx(a0, b0)
            w3, w7 = _cmpx(a3, b3)
            w1, w5 = _cmpx(a1, b1)
            w2, w6 = _cmpx(a2, b2)
            w0, w2 = _cmpx(w0, w2)
            w1, w3 = _cmpx(w1, w3)
            w4, w6 = _cmpx(w4, w6)
            w5, w7 = _cmpx(w5, w7)
            w0, w1 = _cmpx(w0, w1)
            w2, w3 = _cmpx(w2, w3)
            w4, w5 = _cmpx(w4, w5)
            w6, w7 = _cmpx(w6, w7)
            for j, w in enumerate((w0, w1, w2, w3, w4, w5, w6, w7)):
                ref[base + j, :] = w


def _sc_pass_c(ra, rb, R):
    @plsc.parallel_loop(0, R // 16, unroll=1)
    def body(g):
        base = g * 16
        for ref in (ra, rb):
            v = [None] * 16
            for i in range(4):
                i2 = 7 - i
                a_i = _asc(ref[base + i, :])
                a_i2 = _asc(ref[base + i2, :])
                b_i = _desc(ref[base + 8 + i2, :])
                b_i2 = _desc(ref[base + 8 + i, :])
                v[i], v[8 + i] = _cmpx(a_i, b_i)
                v[i2], v[8 + i2] = _cmpx(a_i2, b_i2)
            for h in (4, 2, 1):
                for blk in range(0, 16, 2 * h):
                    for i in range(h):
                        v[blk + i], v[blk + i + h] = _cmpx(v[blk + i],
                                                           v[blk + i + h])
            for j in range(16):
                ref[base + j, :] = v[j]


def _sc_pass_d(ra, rb, R):
    @plsc.parallel_loop(0, R // 32, unroll=1)
    def body(g):
        base = g * 32
        for ref in (ra, rb):
            v = [None] * 32
            for i in range(8):
                i2 = 15 - i
                a_i = _asc(ref[base + i, :])
                a_i2 = _asc(ref[base + i2, :])
                b_i = _desc(ref[base + 16 + i2, :])
                b_i2 = _desc(ref[base + 16 + i, :])
                v[i], v[16 + i] = _cmpx(a_i, b_i)
                v[i2], v[16 + i2] = _cmpx(a_i2, b_i2)
            for h in (8, 4, 2, 1):
                for blk in range(0, 32, 2 * h):
                    for i in range(h):
                        v[blk + i], v[blk + i + h] = _cmpx(v[blk + i],
                                                           v[blk + i + h])
            for j in range(32):
                ref[base + j, :] = v[j]


def _sc_stage1(ra, rb, R, m):
    half = m // 2

    @plsc.parallel_loop(0, (R // (2 * m)) * half, unroll=1)
    def body(t):
        g = t // half
        i = t - g * half
        base = g * (2 * m)
        i2 = m - 1 - i
        for ref in (ra, rb):
            a_i = _asc(ref[base + i, :])
            a_i2 = _asc(ref[base + i2, :])
            b_i = _desc(ref[base + m + i2, :])
            b_i2 = _desc(ref[base + m + i, :])
            ref[base + i, :], ref[base + m + i, :] = _cmpx(a_i, b_i)
            ref[base + i2, :], ref[base + m + i2, :] = _cmpx(a_i2, b_i2)


def _sc_fused3(ra, rb, R, h):
    q = h // 4

    @plsc.parallel_loop(0, (R // (2 * h)) * q, unroll=1)
    def body(t):
        g = t // q
        i = t - g * q
        base = g * 2 * h + i
        for ref in (ra, rb):
            v = [ref[base + j * q, :] for j in range(8)]
            v[0], v[4] = _cmpx(v[0], v[4])
            v[1], v[5] = _cmpx(v[1], v[5])
            v[2], v[6] = _cmpx(v[2], v[6])
            v[3], v[7] = _cmpx(v[3], v[7])
            v[0], v[2] = _cmpx(v[0], v[2])
            v[1], v[3] = _cmpx(v[1], v[3])
            v[4], v[6] = _cmpx(v[4], v[6])
            v[5], v[7] = _cmpx(v[5], v[7])
            v[0], v[1] = _cmpx(v[0], v[1])
            v[2], v[3] = _cmpx(v[2], v[3])
            v[4], v[5] = _cmpx(v[4], v[5])
            v[6], v[7] = _cmpx(v[6], v[7])
            for j in range(8):
                ref[base + j * q, :] = v[j]


def _sc_fused2(ra, rb, R, h):
    q = h // 2

    @plsc.parallel_loop(0, (R // (2 * h)) * q, unroll=1)
    def body(t):
        g = t // q
        i = t - g * q
        base = g * 2 * h + i
        for ref in (ra, rb):
            v0 = ref[base, :]
            v1 = ref[base + q, :]
            v2 = ref[base + 2 * q, :]
            v3 = ref[base + 3 * q, :]
            v0, v2 = _cmpx(v0, v2)
            v1, v3 = _cmpx(v1, v3)
            v0, v1 = _cmpx(v0, v1)
            v2, v3 = _cmpx(v2, v3)
            ref[base, :] = v0
            ref[base + q, :] = v1
            ref[base + 2 * q, :] = v2
            ref[base + 3 * q, :] = v3


def _sc_single(ra, rb, R, h):
    @plsc.parallel_loop(0, R // 2, unroll=4)
    def body(t):
        q = t // h
        blk = q * (2 * h)
        i = t - q * h
        for ref in (ra, rb):
            u = ref[blk + i, :]
            v = ref[blk + i + h, :]
            ref[blk + i, :], ref[blk + i + h, :] = _cmpx(u, v)


def _sc_final(ra, rb, R):
    def body(g, acc):
        base = g * 8
        rows = []
        for ref in (ra, rb):
            v = [ref[base + j, :] for j in range(8)]
            v[0], v[4] = _cmpx(v[0], v[4])
            v[1], v[5] = _cmpx(v[1], v[5])
            v[2], v[6] = _cmpx(v[2], v[6])
            v[3], v[7] = _cmpx(v[3], v[7])
            v[0], v[2] = _cmpx(v[0], v[2])
            v[1], v[3] = _cmpx(v[1], v[3])
            v[4], v[6] = _cmpx(v[4], v[6])
            v[5], v[7] = _cmpx(v[5], v[7])
            v[0], v[1] = _cmpx(v[0], v[1])
            v[2], v[3] = _cmpx(v[2], v[3])
            v[4], v[5] = _cmpx(v[4], v[5])
            v[6], v[7] = _cmpx(v[6], v[7])
            rows.append(v)
        for j in range(8):
            d = _asc(rows[0][j]) - _asc(rows[1][j])
            acc = acc + d * d
        return acc
    return lax.fori_loop(0, R // 8, body, jnp.zeros((16,), jnp.float32),
                         unroll=1)


def _sc_body(xt, yt, th, out, xv, yv, pxv, pyv, px2, py2, thv, outv, *,
             n_proj):
    cid = lax.axis_index("c")
    sid = lax.axis_index("s")
    wid = sid * _NC + cid
    pltpu.sync_copy(th, thv)
    for bi in range(2):
        b = wid * 2 + bi
        pltpu.sync_copy(xt.at[b], xv)
        pltpu.sync_copy(yt.at[b], yv)

        def pbody(pp, _):
            p0 = pp * 2
            p1 = p0 + 1
            t0 = thv[0, p0, :]
            t1 = thv[1, p0, :]
            t2 = thv[2, p0, :]
            u0 = thv[0, p1, :]
            u1 = thv[1, p1, :]
            u2 = thv[2, p1, :]

            @plsc.parallel_loop(0, 128, unroll=1)
            def proj(r):
                x0 = xv[0, r, :]
                x1 = xv[1, r, :]
                x2 = xv[2, r, :]
                y0 = yv[0, r, :]
                y1 = yv[1, r, :]
                y2 = yv[2, r, :]
                pxv[r, :] = x0 * t0 + x1 * t1 + x2 * t2
                pyv[r, :] = y0 * t0 + y1 * t1 + y2 * t2
                px2[r, :] = x0 * u0 + x1 * u1 + x2 * u2
                py2[r, :] = y0 * u0 + y1 * u1 + y2 * u2

            for ra, rb, p in ((pxv, pyv, p0), (px2, py2, p1)):
                _sc_pass_a(ra, rb, 128)
                _sc_pass_b(ra, rb, 128)
                _sc_pass_c(ra, rb, 128)
                _sc_pass_d(ra, rb, 128)
                for m in (32, 64):
                    _sc_stage1(ra, rb, 128, m)
                    if m == 32:
                        _sc_fused3(ra, rb, 128, 16)
                        _sc_fused2(ra, rb, 128, 2)
                    else:
                        _sc_fused3(ra, rb, 128, 32)
                acc = _sc_final(ra, rb, 128)
                outv[p, :] = acc
            return 0
        lax.fori_loop(0, n_proj // 2, pbody, 0)
        pltpu.sync_copy(outv, out.at[b])


def _swd_sc_part(x, y, theta_b):
    b, n, _ = x.shape
    p = theta_b.shape[1]
    xt = x.transpose(0, 2, 1).reshape(b, 3, n // 16, 16)
    yt = y.transpose(0, 2, 1).reshape(b, 3, n // 16, 16)
    mesh = plsc.VectorSubcoreMesh(core_axis_name="c", subcore_axis_name="s")
    f = pl.kernel(
        functools.partial(_sc_body, n_proj=p),
        out_type=jax.ShapeDtypeStruct((b, p, 16), jnp.float32),
        mesh=mesh,
        scratch_types=[
            pltpu.VMEM((3, n // 16, 16), jnp.float32),
            pltpu.VMEM((3, n // 16, 16), jnp.float32),
            pltpu.VMEM((n // 16, 16), jnp.float32),
            pltpu.VMEM((n // 16, 16), jnp.float32),
            pltpu.VMEM((n // 16, 16), jnp.float32),
            pltpu.VMEM((n // 16, 16), jnp.float32),
            pltpu.VMEM((3, p, 16), jnp.float32),
            pltpu.VMEM((p, 16), jnp.float32),
        ],
        compiler_params=pltpu.CompilerParams(needs_layout_passes=False,
                                             use_tc_tiling_on_sc=False),
    )
    s3 = f(xt, yt, theta_b)
    return jnp.sum(s3, axis=2)


def kernel(x, y):
    theta_t = jnp.asarray(_THETA.T)
    theta_sc = jnp.broadcast_to(theta_t[:, :, None], (3, _NUM_PROJS, 16))
    s = _swd_sc_part(x, y, theta_sc)
    return jnp.mean(jnp.sqrt(jnp.mean(s, axis=1)))


# speedup vs baseline: 1.1831x; 1.1831x over previous
import functools

import numpy as np
import jax
import jax.numpy as jnp
from jax.experimental import pallas as pl

_NUM_PROJS = 128


def _sobol2_np(n):
    bits = 30
    ms = [1]
    for k in range(1, bits):
        ms.append((2 * ms[k - 1]) ^ ms[k - 1])
    v0 = [1 << (bits - 1 - k) for k in range(bits)]
    v1 = [ms[k] << (bits - 1 - k) for k in range(bits)]
    x0, x1 = 0, 0
    out = np.zeros((n, 2), dtype=np.float64)
    for i in range(1, n):
        c = 0
        j = i - 1
        while j & 1:
            j >>= 1
            c += 1
        x0 ^= v0[c]
        x1 ^= v1[c]
        out[i, 0] = x0 / float(1 << bits)
        out[i, 1] = x1 / float(1 << bits)
    return out


def _theta_np(num_projs):
    net = _sobol2_np(num_projs)
    alpha = net[:, 0:1]
    tau = net[:, 1:2]
    r = 2.0 * np.sqrt(np.maximum(tau - tau ** 2, 0.0))
    theta = np.concatenate([
        r * np.cos(2.0 * np.pi * alpha),
        r * np.sin(2.0 * np.pi * alpha),
        1.0 - 2.0 * tau,
    ], axis=1)
    return theta.astype(np.float32)


_THETA = _theta_np(_NUM_PROJS)


from jax import lax
from jax.experimental.pallas import tpu as pltpu
from jax.experimental.pallas import tpu_sc as plsc

_NC, _NS = 2, 16


def _asc(v):
    return lax.sort(v, dimension=0, is_stable=False)


def _desc(v):
    return plsc.sort_key_val(v, v, descending=True)[0]


def _cmpx(u, v):
    return jnp.minimum(u, v), jnp.maximum(u, v)


def _sc_pass_a(ra, rb, R):
    @plsc.parallel_loop(0, R // 4, unroll=1)
    def body(g):
        base = g * 4
        for ref in (ra, rb):
            u0 = _asc(ref[base, :])
            u1 = _desc(ref[base + 1, :])
            u2 = _asc(ref[base + 2, :])
            u3 = _desc(ref[base + 3, :])
            lo0, hi0 = _cmpx(u0, u1)
            lo2, hi2 = _cmpx(u2, u3)
            a0 = _asc(lo0)
            a1 = _asc(hi0)
            b0 = _desc(hi2)
            b1 = _desc(lo2)
            w0, w2 = _cmpx(a0, b0)
            w1, w3 = _cmpx(a1, b1)
            o0, o1 = _cmpx(w0, w1)
            o2, o3 = _cmpx(w2, w3)
            ref[base, :] = o0
            ref[base + 1, :] = o1
            ref[base + 2, :] = o2
            ref[base + 3, :] = o3


def _sc_pass_b(ra, rb, R):
    @plsc.parallel_loop(0, R // 8, unroll=1)
    def body(g):
        base = g * 8
        for ref in (ra, rb):
            a0 = _asc(ref[base + 0, :])
            a1 = _asc(ref[base + 1, :])
            a2 = _asc(ref[base + 2, :])
            a3 = _asc(ref[base + 3, :])
            b3 = _desc(ref[base + 4, :])
            b2 = _desc(ref[base + 5, :])
            b1 = _desc(ref[base + 6, :])
            b0 = _desc(ref[base + 7, :])
            w0, w4 = _cmpx(a0, b0)
            w3, w7 = _cmpx(a3, b3)
            w1, w5 = _cmpx(a1, b1)
            w2, w6 = _cmpx(a2, b2)
            w0, w2 = _cmpx(w0, w2)
            w1, w3 = _cmpx(w1, w3)
            w4, w6 = _cmpx(w4, w6)
            w5, w7 = _cmpx(w5, w7)
            w0, w1 = _cmpx(w0, w1)
            w2, w3 = _cmpx(w2, w3)
            w4, w5 = _cmpx(w4, w5)
            w6, w7 = _cmpx(w6, w7)
            for j, w in enumerate((w0, w1, w2, w3, w4, w5, w6, w7)):
                ref[base + j, :] = w


def _sc_pass_c(ra, rb, R):
    @plsc.parallel_loop(0, R // 16, unroll=1)
    def body(g):
        base = g * 16
        for ref in (ra, rb):
            v = [None] * 16
            for i in range(4):
                i2 = 7 - i
                a_i = _asc(ref[base + i, :])
                a_i2 = _asc(ref[base + i2, :])
                b_i = _desc(ref[base + 8 + i2, :])
                b_i2 = _desc(ref[base + 8 + i, :])
                v[i], v[8 + i] = _cmpx(a_i, b_i)
                v[i2], v[8 + i2] = _cmpx(a_i2, b_i2)
            for h in (4, 2, 1):
                for blk in range(0, 16, 2 * h):
                    for i in range(h):
                        v[blk + i], v[blk + i + h] = _cmpx(v[blk + i],
                                                           v[blk + i + h])
            for j in range(16):
                ref[base + j, :] = v[j]


def _sc_stage1(ra, rb, R, m):
    half = m // 2

    @plsc.parallel_loop(0, (R // (2 * m)) * half, unroll=1)
    def body(t):
        g = t // half
        i = t - g * half
        base = g * (2 * m)
        i2 = m - 1 - i
        for ref in (ra, rb):
            a_i = _asc(ref[base + i, :])
            a_i2 = _asc(ref[base + i2, :])
            b_i = _desc(ref[base + m + i2, :])
            b_i2 = _desc(ref[base + m + i, :])
            ref[base + i, :], ref[base + m + i, :] = _cmpx(a_i, b_i)
            ref[base + i2, :], ref[base + m + i2, :] = _cmpx(a_i2, b_i2)


def _sc_fused3(ra, rb, R, h):
    q = h // 4

    @plsc.parallel_loop(0, (R // (2 * h)) * q, unroll=1)
    def body(t):
        g = t // q
        i = t - g * q
        base = g * 2 * h + i
        for ref in (ra, rb):
            v = [ref[base + j * q, :] for j in range(8)]
            v[0], v[4] = _cmpx(v[0], v[4])
            v[1], v[5] = _cmpx(v[1], v[5])
            v[2], v[6] = _cmpx(v[2], v[6])
            v[3], v[7] = _cmpx(v[3], v[7])
            v[0], v[2] = _cmpx(v[0], v[2])
            v[1], v[3] = _cmpx(v[1], v[3])
            v[4], v[6] = _cmpx(v[4], v[6])
            v[5], v[7] = _cmpx(v[5], v[7])
            v[0], v[1] = _cmpx(v[0], v[1])
            v[2], v[3] = _cmpx(v[2], v[3])
            v[4], v[5] = _cmpx(v[4], v[5])
            v[6], v[7] = _cmpx(v[6], v[7])
            for j in range(8):
                ref[base + j * q, :] = v[j]


def _sc_fused2(ra, rb, R, h):
    q = h // 2

    @plsc.parallel_loop(0, (R // (2 * h)) * q, unroll=1)
    def body(t):
        g = t // q
        i = t - g * q
        base = g * 2 * h + i
        for ref in (ra, rb):
            v0 = ref[base, :]
            v1 = ref[base + q, :]
            v2 = ref[base + 2 * q, :]
            v3 = ref[base + 3 * q, :]
            v0, v2 = _cmpx(v0, v2)
            v1, v3 = _cmpx(v1, v3)
            v0, v1 = _cmpx(v0, v1)
            v2, v3 = _cmpx(v2, v3)
            ref[base, :] = v0
            ref[base + q, :] = v1
            ref[base + 2 * q, :] = v2
            ref[base + 3 * q, :] = v3


def _sc_final(ra, rb, R):
    def body(g, acc):
        base = g * 8
        rows = []
        for ref in (ra, rb):
            v = [ref[base + j, :] for j in range(8)]
            v[0], v[4] = _cmpx(v[0], v[4])
            v[1], v[5] = _cmpx(v[1], v[5])
            v[2], v[6] = _cmpx(v[2], v[6])
            v[3], v[7] = _cmpx(v[3], v[7])
            v[0], v[2] = _cmpx(v[0], v[2])
            v[1], v[3] = _cmpx(v[1], v[3])
            v[4], v[6] = _cmpx(v[4], v[6])
            v[5], v[7] = _cmpx(v[5], v[7])
            v[0], v[1] = _cmpx(v[0], v[1])
            v[2], v[3] = _cmpx(v[2], v[3])
            v[4], v[5] = _cmpx(v[4], v[5])
            v[6], v[7] = _cmpx(v[6], v[7])
            rows.append(v)
        for j in range(8):
            d = _asc(rows[0][j]) - _asc(rows[1][j])
            acc = acc + d * d
        return acc
    return lax.fori_loop(0, R // 8, body, jnp.zeros((16,), jnp.float32),
                         unroll=1)


def _sc_body(xt, yt, th, out, xv, yv, pxv, pyv, px2, py2, thv, outv, *,
             n_proj):
    cid = lax.axis_index("c")
    sid = lax.axis_index("s")
    wid = sid * _NC + cid
    pltpu.sync_copy(th, thv)
    for bi in range(2):
        b = wid * 2 + bi
        pltpu.sync_copy(xt.at[b], xv)
        pltpu.sync_copy(yt.at[b], yv)

        def pbody(pp, _):
            p0 = pp * 2
            p1 = p0 + 1
            t0 = thv[0, p0, :]
            t1 = thv[1, p0, :]
            t2 = thv[2, p0, :]
            u0 = thv[0, p1, :]
            u1 = thv[1, p1, :]
            u2 = thv[2, p1, :]

            @plsc.parallel_loop(0, 128, unroll=1)
            def proj(r):
                x0 = xv[0, r, :]
                x1 = xv[1, r, :]
                x2 = xv[2, r, :]
                y0 = yv[0, r, :]
                y1 = yv[1, r, :]
                y2 = yv[2, r, :]
                pxv[r, :] = x0 * t0 + x1 * t1 + x2 * t2
                pyv[r, :] = y0 * t0 + y1 * t1 + y2 * t2
                px2[r, :] = x0 * u0 + x1 * u1 + x2 * u2
                py2[r, :] = y0 * u0 + y1 * u1 + y2 * u2

            for ra, rb, p in ((pxv, pyv, p0), (px2, py2, p1)):
                _sc_pass_a(ra, rb, 128)
                _sc_pass_b(ra, rb, 128)
                _sc_pass_c(ra, rb, 128)
                for m in (16, 32, 64):
                    _sc_stage1(ra, rb, 128, m)
                    if m == 16:
                        _sc_fused2(ra, rb, 128, 8)
                        _sc_fused2(ra, rb, 128, 2)
                    elif m == 32:
                        _sc_fused3(ra, rb, 128, 16)
                        _sc_fused2(ra, rb, 128, 2)
                    else:
                        _sc_fused3(ra, rb, 128, 32)
                acc = _sc_final(ra, rb, 128)
                outv[p, :] = acc
            return 0
        lax.fori_loop(0, n_proj // 2, pbody, 0)
        pltpu.sync_copy(outv, out.at[b])


def _swd_sc_part(x, y, theta_b):
    b, n, _ = x.shape
    p = theta_b.shape[1]
    xt = x.transpose(0, 2, 1).reshape(b, 3, n // 16, 16)
    yt = y.transpose(0, 2, 1).reshape(b, 3, n // 16, 16)
    mesh = plsc.VectorSubcoreMesh(core_axis_name="c", subcore_axis_name="s")
    f = pl.kernel(
        functools.partial(_sc_body, n_proj=p),
        out_type=jax.ShapeDtypeStruct((b, p, 16), jnp.float32),
        mesh=mesh,
        scratch_types=[
            pltpu.VMEM((3, n // 16, 16), jnp.float32),
            pltpu.VMEM((3, n // 16, 16), jnp.float32),
            pltpu.VMEM((n // 16, 16), jnp.float32),
            pltpu.VMEM((n // 16, 16), jnp.float32),
            pltpu.VMEM((n // 16, 16), jnp.float32),
            pltpu.VMEM((n // 16, 16), jnp.float32),
            pltpu.VMEM((3, p, 16), jnp.float32),
            pltpu.VMEM((p, 16), jnp.float32),
        ],
        compiler_params=pltpu.CompilerParams(needs_layout_passes=False,
                                             use_tc_tiling_on_sc=False),
    )
    s3 = f(xt, yt, theta_b)
    return jnp.sum(s3, axis=2)


def kernel(x, y):
    theta_t = jnp.asarray(_THETA.T)
    theta_sc = jnp.broadcast_to(theta_t[:, :, None], (3, _NUM_PROJS, 16))
    s = _swd_sc_part(x, y, theta_sc)
    return jnp.mean(jnp.sqrt(jnp.mean(s, axis=1)))
